# baseline (speedup 1.0000x reference)
import functools

import jax
import jax.numpy as jnp
from jax import lax
from jax.experimental import pallas as pl
from jax.experimental.pallas import tpu as pltpu
from jax.experimental.pallas import tpu_sc as plsc

N_CLS = 1000
DIM = 512
N_CTX = 16
SEQ = 77
SUF_L = SEQ - 1 - N_CTX
B = 256

NBUF = 6
ROWS_W = 16
TOK_ROWS = 3


def _sc_body(cls_hbm, pref_n, pref_p, ctx_n, ctx_p, suf_n, suf_p,
             out_hbm,
             cls_v, b0, b1, b2, b3, b4, b5,
             sg0, sg1, sg2, sg3, sg4, sg5, ss0, ss1, ss2, ss3, ss4, ss5):
    bufs = [b0, b1, b2, b3, b4, b5]
    sems_g = [sg0, sg1, sg2, sg3, sg4, sg5]
    sems_s = [ss0, ss1, ss2, ss3, ss4, ss5]
    nc = 2
    wid = lax.axis_index("s") * nc + lax.axis_index("c")
    half = wid // 16
    j = wid % 16

    pltpu.sync_copy(cls_hbm, cls_v)
    c16 = cls_v[pl.ds(j * ROWS_W, ROWS_W)]
    rowbase = 256 * half + ROWS_W * j

    def do_half(pref_t, ctx_t, suf_t):
        nstatic = 1 + N_CTX

        def suf_src(s):
            return suf_t.at[c16 + N_CLS * (s - 1 - N_CTX)]

        def src(s):
            if s == 0:
                return pref_t.at[c16]
            if s < nstatic:
                return ctx_t.at[c16 * N_CTX + (s - 1)]
            return suf_src(s)

        def fire(s, slot):
            return pltpu.async_copy(src(s), bufs[slot], sems_g[slot])

        def store(s, slot):
            return pltpu.async_copy(
                bufs[slot],
                out_hbm.at[pl.ds(DIM * s + rowbase, ROWS_W)],
                sems_s[slot])

        def wait_store(s, slot):
            pltpu.make_async_copy(
                bufs[slot],
                out_hbm.at[pl.ds(DIM * s + rowbase, ROWS_W)],
                sems_s[slot]).wait()

        for s in range(NBUF):
            fire(s, s)
        for s in range(nstatic):
            slot = s % NBUF
            pltpu.make_async_copy(src(s), bufs[slot], sems_g[slot]).wait()
            store(s, slot)
            if s > 0:
                wait_store(s - 1, (s - 1) % NBUF)
                fire(s - 1 + NBUF, (s - 1) % NBUF)

        def chunk(c, _):
            for k in range(NBUF):
                s = nstatic + c * NBUF + k
                slot = (nstatic + k) % NBUF
                slot_p = (nstatic + k - 1) % NBUF
                pltpu.make_async_copy(
                    suf_src(s), bufs[slot], sems_g[slot]).wait()
                store(s, slot)
                wait_store(s - 1, slot_p)

                @pl.when(s - 1 + NBUF < SEQ)
                def _(s=s, slot_p=slot_p):
                    pltpu.async_copy(
                        suf_src(s - 1 + NBUF), bufs[slot_p], sems_g[slot_p])
            return None

        lax.fori_loop(0, (SEQ - nstatic) // NBUF, chunk, None)
        wait_store(SEQ - 1, (SEQ - 1) % NBUF)

    @pl.when(half == 0)
    def _():
        do_half(pref_n, ctx_n, suf_n)

    @pl.when(half == 1)
    def _():
        do_half(pref_p, ctx_p, suf_p)


def _tok_body(cls_hbm, tokT, tokout_hbm, cls_v, tk_v, orv, sem):
    nc = 2
    wid = lax.axis_index("s") * nc + lax.axis_index("c")
    pltpu.sync_copy(cls_hbm, cls_v)
    for k in range(TOK_ROWS):
        st = wid * TOK_ROWS + k

        @pl.when(st < SEQ)
        def _(st=st):
            pltpu.sync_copy(tokT.at[st], tk_v)
            for i in range(32):
                ci = cls_v[pl.ds(16 * (i % 16), 16)]
                if i >= 16:
                    ci = ci + N_CLS
                orv[pl.ds(16 * i, 16)] = plsc.load_gather(tk_v, [ci])
            pltpu.sync_copy(orv, tokout_hbm.at[st])


def kernel(cls_id, ctx_pos, ctx_neg, token_prefix_pos, token_suffix_pos,
           token_prefix_neg, token_suffix_neg, tokenized_prompts):
    pref_n2 = token_prefix_neg.reshape(N_CLS, DIM)
    pref_p2 = token_prefix_pos.reshape(N_CLS, DIM)
    ctx_n2 = ctx_neg.reshape(N_CLS * N_CTX, DIM)
    ctx_p2 = ctx_pos.reshape(N_CLS * N_CTX, DIM)
    suf_n2 = token_suffix_neg.transpose(1, 0, 2).reshape(N_CLS * SUF_L, DIM)
    suf_p2 = token_suffix_pos.transpose(1, 0, 2).reshape(N_CLS * SUF_L, DIM)
    tokT = tokenized_prompts.transpose(1, 0)

    mesh = plsc.VectorSubcoreMesh(core_axis_name="c", subcore_axis_name="s")
    run = functools.partial(
        pl.kernel,
        mesh=mesh,
        out_type=jax.ShapeDtypeStruct((SEQ * 2 * B, DIM), jnp.float32),
        scratch_types=(
            [pltpu.VMEM((B,), jnp.int32)]
            + [pltpu.VMEM((ROWS_W, DIM), jnp.float32)] * NBUF
            + [pltpu.SemaphoreType.DMA] * (2 * NBUF)
        ),
    )(_sc_body)

    run_tok = functools.partial(
        pl.kernel,
        mesh=mesh,
        compiler_params=pltpu.CompilerParams(needs_layout_passes=False),
        out_type=jax.ShapeDtypeStruct((SEQ, 2 * B), jnp.int32),
        scratch_types=[
            pltpu.VMEM((B,), jnp.int32),
            pltpu.VMEM((2 * N_CLS,), jnp.int32),
            pltpu.VMEM((2 * B,), jnp.int32),
            pltpu.SemaphoreType.DMA,
        ],
    )(_tok_body)

    prompts_flat = run(
        cls_id, pref_n2, pref_p2, ctx_n2, ctx_p2, suf_n2, suf_p2)
    tokT_out = run_tok(cls_id, tokT)
    prompts = prompts_flat.reshape(SEQ, 2 * B, DIM).transpose(1, 0, 2)
    return prompts, tokT_out.transpose(1, 0)


# speedup vs baseline: 1.0685x; 1.0685x over previous
import functools

import jax
import jax.numpy as jnp
from jax import lax
from jax.experimental import pallas as pl
from jax.experimental.pallas import tpu as pltpu
from jax.experimental.pallas import tpu_sc as plsc

N_CLS = 1000
DIM = 512
N_CTX = 16
SEQ = 77
SUF_L = SEQ - 1 - N_CTX
B = 256

NBUF = 6
ROWS_W = 16
TOK_ROWS = 3


def _sc_body(cls_hbm, pref_n, pref_p, ctx_n, ctx_p, suf_n, suf_p, tokT,
             out_hbm, tokout_hbm,
             cls_v, tk_v, orv, b0, b1, b2, b3, b4, b5,
             sg0, sg1, sg2, sg3, sg4, sg5, ss0, ss1, ss2, ss3, ss4, ss5):
    bufs = [b0, b1, b2, b3, b4, b5]
    sems_g = [sg0, sg1, sg2, sg3, sg4, sg5]
    sems_s = [ss0, ss1, ss2, ss3, ss4, ss5]
    nc = 2
    wid = lax.axis_index("s") * nc + lax.axis_index("c")
    half = wid // 16
    j = wid % 16

    pltpu.sync_copy(cls_hbm, cls_v)
    c16 = cls_v[pl.ds(j * ROWS_W, ROWS_W)]
    rowbase = 256 * half + ROWS_W * j

    for k in range(TOK_ROWS):
        st = wid * TOK_ROWS + k

        @pl.when(st < SEQ)
        def _(st=st):
            pltpu.sync_copy(tokT.at[st], tk_v)
            for i in range(32):
                ci = cls_v[pl.ds(16 * (i % 16), 16)]
                if i >= 16:
                    ci = ci + N_CLS
                orv[pl.ds(16 * i, 16)] = plsc.load_gather(tk_v, [ci])
            pltpu.sync_copy(orv, tokout_hbm.at[st])

    def do_half(pref_t, ctx_t, suf_t):
        nstatic = 1 + N_CTX

        def suf_src(s):
            return suf_t.at[c16 + N_CLS * (s - 1 - N_CTX)]

        def src(s):
            if s == 0:
                return pref_t.at[c16]
            if s < nstatic:
                return ctx_t.at[c16 * N_CTX + (s - 1)]
            return suf_src(s)

        def fire(s, slot):
            return pltpu.async_copy(src(s), bufs[slot], sems_g[slot])

        def store(s, slot):
            return pltpu.async_copy(
                bufs[slot],
                out_hbm.at[pl.ds(DIM * s + rowbase, ROWS_W)],
                sems_s[slot])

        for s in range(NBUF):
            fire(s, s)
        for s in range(nstatic):
            slot = s % NBUF
            pltpu.make_async_copy(src(s), bufs[slot], sems_g[slot]).wait()
            store(s, slot).wait()
            fire(s + NBUF, slot)

        def chunk(c, _):
            for k in range(NBUF):
                s = nstatic + c * NBUF + k
                slot = (nstatic + k) % NBUF
                pltpu.make_async_copy(
                    suf_src(s), bufs[slot], sems_g[slot]).wait()
                store(s, slot).wait()

                @pl.when(s + NBUF < SEQ)
                def _(s=s, slot=slot):
                    pltpu.async_copy(
                        suf_src(s + NBUF), bufs[slot], sems_g[slot])
            return None

        lax.fori_loop(0, (SEQ - nstatic) // NBUF, chunk, None)

    @pl.when(half == 0)
    def _():
        do_half(pref_n, ctx_n, suf_n)

    @pl.when(half == 1)
    def _():
        do_half(pref_p, ctx_p, suf_p)


def _tok_body(cls_hbm, tokT, tokout_hbm, cls_v, tk_v, orv, sem):
    nc = 2
    wid = lax.axis_index("s") * nc + lax.axis_index("c")
    pltpu.sync_copy(cls_hbm, cls_v)
    for k in range(TOK_ROWS):
        st = wid * TOK_ROWS + k

        @pl.when(st < SEQ)
        def _(st=st):
            pltpu.sync_copy(tokT.at[st], tk_v)
            for i in range(32):
                ci = cls_v[pl.ds(16 * (i % 16), 16)]
                if i >= 16:
                    ci = ci + N_CLS
                orv[pl.ds(16 * i, 16)] = plsc.load_gather(tk_v, [ci])
            pltpu.sync_copy(orv, tokout_hbm.at[st])


def kernel(cls_id, ctx_pos, ctx_neg, token_prefix_pos, token_suffix_pos,
           token_prefix_neg, token_suffix_neg, tokenized_prompts):
    pref_n2 = token_prefix_neg.reshape(N_CLS, DIM)
    pref_p2 = token_prefix_pos.reshape(N_CLS, DIM)
    ctx_n2 = ctx_neg.reshape(N_CLS * N_CTX, DIM)
    ctx_p2 = ctx_pos.reshape(N_CLS * N_CTX, DIM)
    suf_n2 = token_suffix_neg.transpose(1, 0, 2).reshape(N_CLS * SUF_L, DIM)
    suf_p2 = token_suffix_pos.transpose(1, 0, 2).reshape(N_CLS * SUF_L, DIM)
    tokT = tokenized_prompts.transpose(1, 0)

    mesh = plsc.VectorSubcoreMesh(core_axis_name="c", subcore_axis_name="s")
    run = functools.partial(
        pl.kernel,
        mesh=mesh,
        compiler_params=pltpu.CompilerParams(needs_layout_passes=False),
        out_type=(
            jax.ShapeDtypeStruct((SEQ * 2 * B, DIM), jnp.float32),
            jax.ShapeDtypeStruct((SEQ, 2 * B), jnp.int32),
        ),
        scratch_types=(
            [
                pltpu.VMEM((B,), jnp.int32),
                pltpu.VMEM((2 * N_CLS,), jnp.int32),
                pltpu.VMEM((2 * B,), jnp.int32),
            ]
            + [pltpu.VMEM((ROWS_W, DIM), jnp.float32)] * NBUF
            + [pltpu.SemaphoreType.DMA] * (2 * NBUF)
        ),
    )(_sc_body)

    prompts_flat, tokT_out = run(
        cls_id, pref_n2, pref_p2, ctx_n2, ctx_p2, suf_n2, suf_p2, tokT)
    prompts = prompts_flat.reshape(SEQ, 2 * B, DIM).transpose(1, 0, 2)
    return prompts, tokT_out.transpose(1, 0)
